# baseline (speedup 1.0000x reference)
import jax
import jax.numpy as jnp
from jax import lax
from jax.experimental import pallas as pl
from jax.experimental.pallas import tpu as pltpu
from jax.experimental.pallas import tpu_sc as plsc

NUM_ROWS = 1000000
D = 64
BATCH = 16384
SEQ = 50
B = BATCH * SEQ
NC, NS, L = 2, 16, 16
NW = NC * NS
BT = BATCH // 128
BT_PER_W = BT // NW
BPW = B // NW
NCHUNK = BT_PER_W * SEQ
NBUF = 4
NROUND = NCHUNK // NBUF
TROW = 141


def _body(idx_hbm, table_hbm, out_hbm, ibt_v, lists_v, *rest):
    gbufs = rest[:NBUF]
    tbufs = rest[NBUF:2 * NBUF]
    gsems = rest[2 * NBUF:3 * NBUF]
    ssems = rest[3 * NBUF:4 * NBUF]

    wid = lax.axis_index("s") * NC + lax.axis_index("c")
    bt0 = wid * BT_PER_W

    iota = lax.iota(jnp.int32, L)
    iota50 = iota * SEQ

    def build_bt(bt, acc):
        pltpu.sync_copy(
            idx_hbm.at[pl.ds((bt0 + bt) * 128 * SEQ, 128 * SEQ)], ibt_v)

        def build_s(s, acc2):
            for blv in range(8):
                v = plsc.load_gather(ibt_v, [iota50 + (s + blv * 16 * SEQ)])
                lists_v[pl.ds((bt * SEQ + s) * 128 + blv * 16, L)] = v
                acc2 = acc2 | (v == 0)
            return acc2

        return lax.fori_loop(0, SEQ, build_s, acc)

    acc = lax.fori_loop(0, BT_PER_W, build_bt, jnp.zeros((L,), jnp.bool_))
    has_zero = plsc.all_reduce_population_count(acc)[0] > 0

    def gather(c, k):
        return pltpu.make_async_copy(
            table_hbm.at[lists_v.at[pl.ds(c * 128, 128)]], gbufs[k], gsems[k])

    def store(c, k):
        s = lax.rem(c, SEQ)
        btg = bt0 + lax.div(c, SEQ)
        return pltpu.make_async_copy(
            tbufs[k].at[:, :, pl.ds(0, 128)], out_hbm.at[s, :, btg],
            ssems[k])

    def fix_padding(c, k):
        def do_fix(k=k):
            zeros = jnp.zeros((L,), jnp.float32)

            def blv_body(blv, carry):
                m = lists_v[pl.ds(c * 128 + blv * 16, L)] == 0
                rows = iota + blv * 16

                def col_body(col, carry2):
                    plsc.store_scatter(
                        gbufs[k], [rows, jnp.full((L,), col, jnp.int32)],
                        zeros, mask=m)
                    return carry2

                lax.fori_loop(0, D, col_body, 0)
                return carry

            lax.fori_loop(0, 8, blv_body, 0)

        lax.cond(has_zero, do_fix, lambda: None)

    dt_vecs = [lax.div(iota + c0, 8) for c0 in range(0, D, L)]
    dl_vecs = [lax.rem(iota + c0, 8) for c0 in range(0, D, L)]

    def transpose(k):
        gb, tb = gbufs[k], tbufs[k]

        @plsc.parallel_loop(0, 128, unroll=2)
        def _t(bl):
            blvec = jnp.full((L,), bl, jnp.int32)
            for i, c0 in enumerate(range(0, D, L)):
                vals = plsc.load_gather(gb, [blvec, iota + c0])
                plsc.store_scatter(tb, [dt_vecs[i], dl_vecs[i], blvec], vals)

    def process(c, k):
        gather(c, k).wait()
        fix_padding(c, k)
        cprev = lax.rem(c - NBUF + NCHUNK, NCHUNK)
        store(cprev, k).wait()
        transpose(k)
        store(c, k).start()
        gather(lax.rem(c + NBUF, NCHUNK), k).start()

    for k in range(NBUF):
        gather(k, k).start()
        store(k, k).start()

    def round_body(r, carry):
        for k in range(NBUF):
            process(r * NBUF + k, k)
        return carry

    lax.fori_loop(0, NROUND, round_body, 0)

    for k in range(NBUF):
        store(NCHUNK - NBUF + k, k).wait()
        gather(k, k).wait()


_run = pl.kernel(
    _body,
    out_type=jax.ShapeDtypeStruct((SEQ, 8, BT, 8, 128), jnp.float32),
    mesh=plsc.VectorSubcoreMesh(core_axis_name="c", subcore_axis_name="s"),
    compiler_params=pltpu.CompilerParams(
        needs_layout_passes=False, use_tc_tiling_on_sc=False,
        skip_device_barrier=True),
    scratch_types=(
        [pltpu.VMEM((128 * SEQ,), jnp.int32), pltpu.VMEM((BPW,), jnp.int32)]
        + [pltpu.VMEM((128, D), jnp.float32) for _ in range(NBUF)]
        + [pltpu.VMEM((8, 8, TROW), jnp.float32) for _ in range(NBUF)]
        + [pltpu.SemaphoreType.DMA for _ in range(2 * NBUF)]
    ),
)


def kernel(indices, table):
    assert indices.shape == (BATCH, SEQ) and table.shape == (NUM_ROWS, D)
    idx = indices.reshape(-1).astype(jnp.int32)
    out = _run(idx, table)
    return out.transpose((2, 4, 0, 1, 3)).reshape(BATCH, SEQ, D)


# speedup vs baseline: 1.0099x; 1.0099x over previous
import jax
import jax.numpy as jnp
from jax import lax
from jax.experimental import pallas as pl
from jax.experimental.pallas import tpu as pltpu
from jax.experimental.pallas import tpu_sc as plsc

NUM_ROWS = 1000000
D = 64
BATCH = 16384
SEQ = 50
B = BATCH * SEQ
NC, NS, L = 2, 16, 16
NW = NC * NS
BT = BATCH // 128
BT_PER_W = BT // NW
BPW = B // NW
NCHUNK = BT_PER_W * SEQ
NBUF = 4
NROUND = NCHUNK // NBUF
TROW = 141


def _body(idx_hbm, table_hbm, out_hbm, ibt_v, lists_v, tskew, *rest):
    gbufs = rest[:NBUF]
    cbufs = rest[NBUF:2 * NBUF]
    gsems = rest[2 * NBUF:3 * NBUF]
    ssems = rest[3 * NBUF:4 * NBUF]

    wid = lax.axis_index("s") * NC + lax.axis_index("c")
    bt0 = wid * BT_PER_W

    iota = lax.iota(jnp.int32, L)
    iota50 = iota * SEQ

    def build_bt(bt, acc):
        pltpu.sync_copy(
            idx_hbm.at[pl.ds((bt0 + bt) * 128 * SEQ, 128 * SEQ)], ibt_v)

        def build_s(s, acc2):
            for blv in range(8):
                v = plsc.load_gather(ibt_v, [iota50 + (s + blv * 16 * SEQ)])
                lists_v[pl.ds((bt * SEQ + s) * 128 + blv * 16, L)] = v
                acc2 = acc2 | (v == 0)
            return acc2

        return lax.fori_loop(0, SEQ, build_s, acc)

    acc = lax.fori_loop(0, BT_PER_W, build_bt, jnp.zeros((L,), jnp.bool_))
    has_zero = plsc.all_reduce_population_count(acc)[0] > 0

    def gather(c, k):
        return pltpu.make_async_copy(
            table_hbm.at[lists_v.at[pl.ds(c * 128, 128)]], gbufs[k], gsems[k])

    def store(c, k):
        s = lax.rem(c, SEQ)
        btg = bt0 + lax.div(c, SEQ)
        return pltpu.make_async_copy(
            cbufs[k], out_hbm.at[s, :, btg], ssems[k])

    def fix_padding(c, k):
        def do_fix(k=k):
            zeros = jnp.zeros((L,), jnp.float32)

            def blv_body(blv, carry):
                m = lists_v[pl.ds(c * 128 + blv * 16, L)] == 0
                rows = iota + blv * 16

                def col_body(col, carry2):
                    plsc.store_scatter(
                        gbufs[k], [rows, jnp.full((L,), col, jnp.int32)],
                        zeros, mask=m)
                    return carry2

                lax.fori_loop(0, D, col_body, 0)
                return carry

            lax.fori_loop(0, 8, blv_body, 0)

        lax.cond(has_zero, do_fix, lambda: None)

    dpos_vecs = [(iota + c0) * TROW for c0 in range(0, D, L)]

    def transpose(k):
        gb = gbufs[k]

        @plsc.parallel_loop(0, 128, unroll=2)
        def _t(bl):
            blvec = jnp.full((L,), bl, jnp.int32)
            for i, c0 in enumerate(range(0, D, L)):
                vals = plsc.load_gather(gb, [blvec, iota + c0])
                plsc.store_scatter(tskew, [dpos_vecs[i] + blvec], vals)

        cb = cbufs[k]

        @plsc.parallel_loop(0, D, unroll=2)
        def _c(j):
            av = jnp.full((L,), lax.div(j, 8), jnp.int32)
            bv = jnp.full((L,), lax.rem(j, 8), jnp.int32)
            for m in range(8):
                vals = tskew[pl.ds(j * TROW + m * 16, L)]
                plsc.store_scatter(cb, [av, bv, iota + m * 16], vals)

    def process(c, k):
        gather(c, k).wait()
        fix_padding(c, k)
        cprev = lax.rem(c - NBUF + NCHUNK, NCHUNK)
        store(cprev, k).wait()
        transpose(k)
        store(c, k).start()
        gather(lax.rem(c + NBUF, NCHUNK), k).start()

    for k in range(NBUF):
        gather(k, k).start()
        store(k, k).start()

    def round_body(r, carry):
        for k in range(NBUF):
            process(r * NBUF + k, k)
        return carry

    lax.fori_loop(0, NROUND, round_body, 0)

    for k in range(NBUF):
        store(NCHUNK - NBUF + k, k).wait()
        gather(k, k).wait()


_run = pl.kernel(
    _body,
    out_type=jax.ShapeDtypeStruct((SEQ, 8, BT, 8, 128), jnp.float32),
    mesh=plsc.VectorSubcoreMesh(core_axis_name="c", subcore_axis_name="s"),
    compiler_params=pltpu.CompilerParams(
        needs_layout_passes=False, use_tc_tiling_on_sc=False,
        skip_device_barrier=True),
    scratch_types=(
        [pltpu.VMEM((128 * SEQ,), jnp.int32), pltpu.VMEM((BPW,), jnp.int32),
         pltpu.VMEM((D * TROW,), jnp.float32)]
        + [pltpu.VMEM((128, D), jnp.float32) for _ in range(NBUF)]
        + [pltpu.VMEM((8, 8, 128), jnp.float32) for _ in range(NBUF)]
        + [pltpu.SemaphoreType.DMA for _ in range(2 * NBUF)]
    ),
)


def kernel(indices, table):
    assert indices.shape == (BATCH, SEQ) and table.shape == (NUM_ROWS, D)
    idx = indices.reshape(-1).astype(jnp.int32)
    out = _run(idx, table)
    return out.transpose((2, 4, 0, 1, 3)).reshape(BATCH, SEQ, D)
